# Initial kernel scaffold; baseline (speedup 1.0000x reference)
import jax
import jax.numpy as jnp
from jax.experimental import pallas as pl


def kernel(x, adj_t, adj_t2):
    raise NotImplementedError("write your pallas kernel here")


# speedup vs baseline: 4.8802x; 4.8802x over previous
import functools

import jax
import jax.numpy as jnp
from jax import lax
from jax.experimental import pallas as pl
from jax.experimental.pallas import tpu as pltpu
from jax.experimental.pallas import tpu_sc as plsc

N = 10000
D = 128
E1 = 320000
E2 = 640000
NS = 16
K = 80
N_PAD = 10240
ROWS_PER_TILE = N_PAD // NS
LAST_ROWS = N - 15 * ROWS_PER_TILE


def _sc_body(x_hbm, row1, col1, row2, col2, zeros_hbm, out_hbm,
             acc, col_v, row_v, rows_v, sem):
    c = lax.axis_index("c")
    s = lax.axis_index("s")
    rbase = s * ROWS_PER_TILE

    pltpu.sync_copy(zeros_hbm, acc.at[pl.ds(rbase, ROWS_PER_TILE)])
    plsc.subcore_barrier()

    def edge_loop(row_hbm, col_hbm, n_edges):
        per_tile = n_edges // NS
        n_chunks = per_tile // K
        tbase = s * per_tile

        def body(j, carry):
            base = tbase + j * K
            pltpu.sync_copy(col_hbm.at[pl.ds(base, K)], col_v)
            pltpu.sync_copy(row_hbm.at[pl.ds(base, K)], row_v)
            pltpu.async_copy(x_hbm.at[col_v], rows_v, sem).wait()
            pltpu.sync_copy(rows_v, acc.at[row_v], add=True)
            return carry

        lax.fori_loop(0, n_chunks, body, 0)

    @pl.when(c == 0)
    def _():
        edge_loop(row1, col1, E1)

    @pl.when(c == 1)
    def _():
        edge_loop(row2, col2, E2)

    plsc.subcore_barrier()

    def writeout(col0):
        @pl.when(s < NS - 1)
        def _():
            pltpu.sync_copy(
                acc.at[pl.ds(rbase, ROWS_PER_TILE)],
                out_hbm.at[pl.ds(rbase, ROWS_PER_TILE), pl.ds(col0, D)])

        @pl.when(s == NS - 1)
        def _():
            pltpu.sync_copy(
                acc.at[pl.ds((NS - 1) * ROWS_PER_TILE, LAST_ROWS)],
                out_hbm.at[pl.ds((NS - 1) * ROWS_PER_TILE, LAST_ROWS),
                           pl.ds(col0, D)])

    @pl.when(c == 0)
    def _():
        writeout(0)

    @pl.when(c == 1)
    def _():
        writeout(D)


@jax.jit
def kernel(x, adj_t, adj_t2):
    row1, col1 = adj_t[0], adj_t[1]
    row2, col2 = adj_t2[0], adj_t2[1]
    zeros = jnp.zeros((ROWS_PER_TILE, D), jnp.float32)
    mesh = plsc.VectorSubcoreMesh(core_axis_name="c", subcore_axis_name="s")
    f = pl.kernel(
        _sc_body,
        out_type=jax.ShapeDtypeStruct((N, 2 * D), jnp.float32),
        mesh=mesh,
        scratch_types=[
            pltpu.VMEM_SHARED((N_PAD, D), jnp.float32),
            pltpu.VMEM((K,), jnp.int32),
            pltpu.VMEM((K,), jnp.int32),
            pltpu.VMEM((K, D), jnp.float32),
            pltpu.SemaphoreType.DMA,
        ],
    )
    return f(x, row1, col1, row2, col2, zeros)
